# Initial kernel scaffold; baseline (speedup 1.0000x reference)
import jax
import jax.numpy as jnp
from jax.experimental import pallas as pl


def kernel(h, tag_positions, char_tag_id, tag_ids_set, codebook, W_ih, W_hh, b_ih, b_hh, W_inj):
    raise NotImplementedError("write your pallas kernel here")


# speedup vs baseline: 45.2658x; 45.2658x over previous
import functools

import jax
import jax.numpy as jnp
from jax import lax
from jax.experimental import pallas as pl
from jax.experimental.pallas import tpu as pltpu
from jax.experimental.pallas import tpu_sc as plsc

D = 1024
P_PAD = 72
N_CELL = 16
HI = jax.lax.Precision.HIGHEST

_c11 = (((1,), (1,)), ((), ()))


def _tc_body(h4_ref, cb_ref, cbe_ref, wih_ref, whh_ref, bih_ref, bhh_ref,
             winj_ref, idx_ref, cm_ref, inj_ref):
    h4 = h4_ref[...]
    cb = cb_ref[...]
    cbe = cbe_ref[...]
    ones = jnp.ones((1, D), jnp.float32)
    cb_n = lax.dot_general(ones, cb * cb, _c11, precision=HI)

    gi = lax.dot_general(h4, wih_ref[...], _c11, precision=HI)
    gh = lax.dot_general(cbe, whh_ref[...], _c11, precision=HI)
    bih = bih_ref[...]
    bhh = bhh_ref[...]
    gir = gi[:, :D] + bih[0:1, :]
    giz = gi[:, D:2 * D] + bih[1:2, :]
    gin = gi[:, 2 * D:] + bih[2:3, :]
    ghr = gh[:, :D] + bhh[0:1, :]
    ghz = gh[:, D:2 * D] + bhh[1:2, :]
    ghn = gh[:, 2 * D:] + bhh[2:3, :]

    r3 = jax.nn.sigmoid(gir[:, None, :] + ghr[None, :, :])
    z3 = jax.nn.sigmoid(giz[:, None, :] + ghz[None, :, :])
    n3 = jnp.tanh(gin[:, None, :] + r3 * ghn[None, :, :])
    new3 = (1.0 - z3) * n3 + z3 * cbe[None, :, :]
    p_iota = lax.broadcasted_iota(jnp.int32, (N_CELL, P_PAD, D), 1)
    new3 = jnp.where(p_iota == 65, h4[:, None, :], new3)

    newf = new3.reshape(N_CELL * P_PAD, D)
    score = cb_n - 2.0 * lax.dot_general(newf, cb, _c11, precision=HI)
    nn = jnp.sum(newf * newf, axis=1, keepdims=True)
    idx_ref[...] = jnp.argmin(score, axis=1, keepdims=True).astype(jnp.int32)
    cm_ref[...] = (nn + jnp.min(score, axis=1, keepdims=True)) * (1.0 / D)

    inj_ref[0:64, :] = lax.dot_general(cb, winj_ref[...], _c11, precision=HI)
    inj_ref[64:128, :] = jnp.zeros((64, D), jnp.float32)


def _tc_tables(h4, cb, cbe, W_ih, W_hh, bih3, bhh3, W_inj):
    return pl.pallas_call(
        _tc_body,
        out_shape=[
            jax.ShapeDtypeStruct((N_CELL * P_PAD, 1), jnp.int32),
            jax.ShapeDtypeStruct((N_CELL * P_PAD, 1), jnp.float32),
            jax.ShapeDtypeStruct((128, D), jnp.float32),
        ],
    )(h4, cb, cbe, W_ih, W_hh, bih3, bhh3, W_inj)


def _sc_body(tags_hbm, idx_hbm, cm_hbm, char_hbm, inj_hbm, out_hbm, avg_hbm,
             tags_v, idx_v, cm_v, char_v, act_v, slots_v, fin_v, rows_v,
             avg_v, sem):
    wid = lax.axis_index("s") * 2 + lax.axis_index("c")

    @pl.when(wid == 0)
    def _():
        pltpu.sync_copy(tags_hbm, tags_v)
        pltpu.sync_copy(idx_hbm, idx_v)
        pltpu.sync_copy(cm_hbm, cm_v)
        pltpu.sync_copy(char_hbm, char_v)
        zeros16 = jnp.zeros((16,), jnp.int32)
        act_v[...] = zeros16
        slots_v[...] = zeros16
        fin_v[...] = jnp.full((16,), 64, jnp.int32)
        char = char_v[...]
        lane = lax.broadcasted_iota(jnp.int32, (16,), 0)
        m0 = lane == 0

        def step(t, carry):
            tc, nu = carry
            tsp = jnp.full((16,), t, jnp.int32)
            b = plsc.load_gather(tags_v, [tsp])
            s = plsc.load_gather(tags_v, [tsp + 128])
            tok = plsc.load_gather(tags_v, [tsp + 256])
            cell = b * 4 + s
            act = plsc.load_gather(act_v, [b])
            is_char = tok == char
            has_act = act > 0
            slot_b = jnp.where(has_act, (act - 1) & 3, 0)
            p = plsc.load_gather(slots_v, [b * 4 + slot_b])
            flat = jnp.where(is_char, cell * P_PAD + 65, cell * P_PAD + p)
            code = plsc.load_gather(idx_v, [flat])
            cm = plsc.load_gather(cm_v, [flat])
            did = jnp.logical_or(is_char, has_act)
            slot_u = jnp.where(is_char, act & 3, slot_b)
            plsc.store_scatter(slots_v, [b * 4 + slot_u], code + 1,
                               mask=jnp.logical_and(m0, did))
            ich = is_char.astype(jnp.int32)
            plsc.store_scatter(act_v, [b], act + ich, mask=m0)
            tc = tc + jnp.where(did, cm, 0.0)
            nu = nu + jnp.where(did, 1, 0)
            act2 = act + ich
            inj_code = plsc.load_gather(slots_v, [b * 4 + ((act2 - 1) & 3)]) - 1
            plsc.store_scatter(fin_v, [cell], inj_code,
                               mask=jnp.logical_and(m0, act2 > 0))
            return tc, nu

        tc, nu = lax.fori_loop(
            0, 128, step,
            (jnp.zeros((16,), jnp.float32), jnp.zeros((16,), jnp.int32)))
        avg_v[...] = tc / jnp.maximum(nu, 1).astype(jnp.float32)
        pltpu.sync_copy(avg_v, avg_hbm)
        pltpu.async_copy(inj_hbm.at[fin_v], rows_v, sem).wait()
        pltpu.sync_copy(rows_v, out_hbm)


def _sc_automaton(tags, idxf, cmf, charv, inj_tab):
    mesh = plsc.VectorSubcoreMesh(core_axis_name="c", subcore_axis_name="s")
    run = functools.partial(
        pl.kernel, _sc_body, mesh=mesh,
        compiler_params=pltpu.CompilerParams(needs_layout_passes=False),
        out_type=[
            jax.ShapeDtypeStruct((N_CELL, D), jnp.float32),
            jax.ShapeDtypeStruct((16,), jnp.float32),
        ],
        scratch_types=[
            pltpu.VMEM((512,), jnp.int32),
            pltpu.VMEM((N_CELL * P_PAD,), jnp.int32),
            pltpu.VMEM((N_CELL * P_PAD,), jnp.float32),
            pltpu.VMEM((16,), jnp.int32),
            pltpu.VMEM((16,), jnp.int32),
            pltpu.VMEM((16,), jnp.int32),
            pltpu.VMEM((16,), jnp.int32),
            pltpu.VMEM((N_CELL, D), jnp.float32),
            pltpu.VMEM((16,), jnp.float32),
            pltpu.SemaphoreType.DMA,
        ],
    )()
    return run(tags, idxf, cmf, charv, inj_tab)


def kernel(h, tag_positions, char_tag_id, tag_ids_set, codebook, W_ih, W_hh,
           b_ih, b_hh, W_inj):
    B, T, d = h.shape
    h4 = h[:, :4, :].reshape(N_CELL, d).astype(jnp.float32)
    cb = codebook.astype(jnp.float32)
    cbe = jnp.zeros((P_PAD, d), jnp.float32).at[1:65].set(cb)
    tags = jnp.zeros((4, 128), jnp.int32).at[:3].set(
        tag_positions.astype(jnp.int32).T).reshape(512)
    charv = jnp.full((16,), char_tag_id, jnp.int32)
    bih3 = b_ih.reshape(3, d).astype(jnp.float32)
    bhh3 = b_hh.reshape(3, d).astype(jnp.float32)

    idxf, cmf, inj_tab = _tc_tables(h4, cb, cbe, W_ih.astype(jnp.float32),
                                    W_hh.astype(jnp.float32), bih3, bhh3,
                                    W_inj.astype(jnp.float32))
    block, avg = _sc_automaton(tags, idxf.reshape(N_CELL * P_PAD),
                               cmf.reshape(N_CELL * P_PAD), charv, inj_tab)
    injection = jnp.zeros((B, T, d), h.dtype).at[:, :4, :].set(
        block.reshape(B, 4, d))
    return injection, avg[0]
